# baseline (speedup 1.0000x reference)
import functools

import jax
import jax.numpy as jnp
from jax import lax
from jax.experimental import pallas as pl
from jax.experimental.pallas import tpu as pltpu
from jax.experimental.pallas import tpu_sc as plsc

DIM = 64
LANES = 16
NUM_CORES = 2
NUM_SUBCORES = 16
NUM_WORKERS = NUM_CORES * NUM_SUBCORES

CHUNK = 128
NBUF = 3

PACK = 128 // DIM
TC_BLK = 256


def _tc_pack_table(vocab):
    half = vocab // 2
    nblk = half // TC_BLK
    out_rows = (nblk + 1) * TC_BLK

    def body(lo_ref, hi_ref, out_ref):
        out_ref[:, :DIM] = lo_ref[...].T
        out_ref[:, DIM:] = hi_ref[...].T

    return pl.pallas_call(
        body,
        grid=(nblk,),
        in_specs=[pl.BlockSpec((DIM, TC_BLK), lambda i: (0, i)),
                  pl.BlockSpec((DIM, TC_BLK), lambda i: (0, i + nblk))],
        out_specs=pl.BlockSpec((TC_BLK, 128), lambda i: (i, 0)),
        out_shape=jax.ShapeDtypeStruct((out_rows, 128), jnp.float32),
    ), half


def _sc_pack_table(vocab):
    nblk = vocab // 128
    per_w, rem = divmod(nblk, NUM_WORKERS)

    mesh = plsc.VectorSubcoreMesh(
        core_axis_name="c", subcore_axis_name="s",
        num_cores=NUM_CORES, num_subcores=NUM_SUBCORES)

    @functools.partial(
        pl.kernel,
        out_type=jax.ShapeDtypeStruct(((vocab + 128) // PACK, 128),
                                      jnp.float32),
        mesh=mesh,
        compiler_params=pltpu.CompilerParams(use_tc_tiling_on_sc=True,
                                             needs_layout_passes=False),
        scratch_types=(
            [pltpu.VMEM((DIM, 128), jnp.float32) for _ in range(NBUF)]
            + [pltpu.VMEM((DIM, 128), jnp.float32) for _ in range(NBUF)]
            + [pltpu.SemaphoreType.DMA for _ in range(2 * NBUF)]
        ),
    )
    def k(src_hbm, out_hbm, i0, i1, i2, o0, o1, o2, g0, g1, g2, s0, s1, s2):
        wid = lax.axis_index("s") * NUM_CORES + lax.axis_index("c")
        start = wid * per_w + jnp.minimum(wid, rem)
        count = per_w + jnp.where(wid < rem, 1, 0)
        tins = (i0, i1, i2)
        touts = (o0, o1, o2)
        gsems = (g0, g1, g2)
        ssems = (s0, s1, s2)
        iota = lax.iota(jnp.int32, LANES)
        zeros = jnp.zeros((LANES,), jnp.int32)

        def g_desc(blk, b):
            src = src_hbm.at[:, pl.ds(blk * 128, 128)]
            return pltpu.make_async_copy(src, tins[b], gsems[b])

        def s_desc(blk, b):
            dst = out_hbm.at[pl.ds(blk * DIM, DIM)]
            return pltpu.make_async_copy(touts[b], dst, ssems[b])

        def xpose(b):
            tin, tout = tins[b], touts[b]

            @plsc.parallel_loop(0, 128 // PACK, step=1, unroll=1)
            def _pair(v2):
                for par in range(PACK):
                    col = zeros + (PACK * v2 + par)
                    for q in range(DIM // LANES):
                        vals = plsc.load_gather(tin, [iota + q * LANES, col])
                        tout[v2, pl.ds(par * DIM + q * LANES, LANES)] = vals

        def step(i, b, first, last):
            nxt = (b + 1) % NBUF
            if not first:
                s_desc(start + i - 2, nxt).wait()

            def prefetch():
                g_desc(start + i + 1, nxt).start()
            if last:
                pl.when(i + 1 < count)(prefetch)
            else:
                prefetch()
            g_desc(start + i, b).wait()
            xpose(b)
            s_desc(start + i, b).start()

        g_desc(start, 0).start()
        step(0, 0, first=True, last=False)
        step(1, 1, first=True, last=False)

        def body3(t, carry):
            for kk in range(NBUF):
                j = NBUF * t + (2 + kk)
                step(j, (2 + kk) % NBUF, first=False, last=False)
            return carry
        nmid = (count - 2 - 3) // NBUF
        lax.fori_loop(0, nmid, body3, 0)

        done = 2 + NBUF * nmid

        def tail_step(j):
            for b in range(NBUF):
                pl.when(j % NBUF == b)(
                    lambda b=b: step(j, b, first=False, last=True))
        for off in range(5):
            jj = done + off
            pl.when(jj < count)(lambda jj=jj: tail_step(jj))

        def drain(j):
            for b in range(NBUF):
                pl.when(j % NBUF == b)(
                    lambda b=b: s_desc(start + j, b).wait())
        drain(count - 2)
        drain(count - 1)

    return k


def _sc_embed_add(n_rows, pos_vocab, vocab_main, n_tail, half):
    assert n_rows % (NUM_WORKERS * CHUNK) == 0
    rows_per_w = n_rows // NUM_WORKERS
    chunks = rows_per_w // CHUNK
    assert chunks >= 5 and (chunks - 2) % NBUF == 0
    small_rows = pos_vocab + n_tail

    mesh = plsc.VectorSubcoreMesh(
        core_axis_name="c", subcore_axis_name="s",
        num_cores=NUM_CORES, num_subcores=NUM_SUBCORES)

    @functools.partial(
        pl.kernel,
        out_type=jax.ShapeDtypeStruct((n_rows, DIM), jnp.float32),
        mesh=mesh,
        compiler_params=pltpu.CompilerParams(use_tc_tiling_on_sc=False,
                                             needs_layout_passes=False),
        scratch_types=(
            [pltpu.VMEM((rows_per_w,), jnp.int32),
             pltpu.VMEM((rows_per_w,), jnp.int32),
             pltpu.VMEM((rows_per_w,), jnp.int32),
             pltpu.VMEM((small_rows * DIM,), jnp.float32)]
            + [pltpu.VMEM((CHUNK, DIM), jnp.float32) for _ in range(NBUF)]
            + [pltpu.SemaphoreType.DMA for _ in range(2 * NBUF)]
        ),
    )
    def k(x_hbm, pos_hbm, tok_hbm, posemb_hbm, out_hbm,
          idx_t, idx_p, idx_g, pos_tab, b0, b1, b2, g0, g1, g2, s0, s1, s2):
        wid = lax.axis_index("s") * NUM_CORES + lax.axis_index("c")
        base = wid * rows_per_w
        pltpu.sync_copy(x_hbm.at[pl.ds(base, rows_per_w)], idx_t)
        pltpu.sync_copy(pos_hbm.at[pl.ds(base, rows_per_w)], idx_p)
        pltpu.sync_copy(posemb_hbm, pos_tab)

        @plsc.parallel_loop(0, rows_per_w // LANES, step=1, unroll=4)
        def _remap(g):
            sl = pl.ds(g * LANES, LANES)
            t = idx_t[sl]
            t2 = t + t
            idx_g[sl] = jnp.where(t < half, t2, t2 - (2 * half - 1))

        bufs = (b0, b1, b2)
        gsems = (g0, g1, g2)
        ssems = (s0, s1, s2)
        iota = lax.iota(jnp.int32, LANES)
        dnums = lax.GatherDimensionNumbers(
            offset_dims=(), collapsed_slice_dims=(0,), start_index_map=(0,))

        def g_desc(c, b):
            src = tok_hbm.at[idx_g.at[pl.ds(c * CHUNK, CHUNK)]]
            return pltpu.make_async_copy(src, bufs[b], gsems[b])

        def s_desc(c, b):
            dst = out_hbm.at[pl.ds(base + c * CHUNK, CHUNK)]
            return pltpu.make_async_copy(bufs[b], dst, ssems[b])

        zeros = jnp.zeros((LANES,), jnp.int32)

        def add_chunk(c, b):
            buf = bufs[b]

            mx = idx_t[pl.ds(c * CHUNK, LANES)]
            for g in range(1, CHUNK // LANES):
                mx = jnp.maximum(mx, idx_t[pl.ds(c * CHUNK + g * LANES,
                                                 LANES)])
            has_tail = lax.reduce_max(mx, (0,)) >= vocab_main

            def fast():
                @plsc.parallel_loop(0, CHUNK, step=1, unroll=2)
                def _row(r):
                    pidx = plsc.load_gather(idx_p, [zeros + (c * CHUNK + r)])
                    rbase = pidx * DIM
                    for q in range(DIM // LANES):
                        addr = rbase + (iota + q * LANES)
                        pv = plsc.load_gather(pos_tab, [addr])
                        sl = pl.ds(q * LANES, LANES)
                        buf[r, sl] = buf[r, sl] + pv

            def slow():
                @plsc.parallel_loop(0, CHUNK, step=1, unroll=1)
                def _row(r):
                    pidx = plsc.load_gather(idx_p, [zeros + (c * CHUNK + r)])
                    tidx = plsc.load_gather(idx_t, [zeros + (c * CHUNK + r)])
                    rbase = pidx * DIM
                    is_tail = tidx >= vocab_main
                    tbase = (jnp.maximum(tidx - vocab_main, 0)
                             + pos_vocab) * DIM
                    for q in range(DIM // LANES):
                        off = iota + q * LANES
                        pv = plsc.load_gather(pos_tab, [rbase + off])
                        tv = plsc.load_gather(pos_tab, [tbase + off])
                        sl = pl.ds(q * LANES, LANES)
                        tok = jnp.where(is_tail, tv, buf[r, sl])
                        buf[r, sl] = tok + pv

            pl.when(has_tail)(slow)
            pl.when(jnp.logical_not(has_tail))(fast)

        def step(j, b, first, last):
            nxt = (b + 1) % NBUF
            if not first:
                s_desc(j - 2, nxt).wait()
            if not last:
                g_desc(j + 1, nxt).start()
            g_desc(j, b).wait()
            add_chunk(j, b)
            s_desc(j, b).start()

        g_desc(0, 0).start()
        step(0, 0, first=True, last=False)
        step(1, 1, first=True, last=False)

        def body3(t, carry):
            for kk in range(NBUF):
                j = NBUF * t + (2 + kk)
                step(j, (2 + kk) % NBUF, first=False, last=False)
            return carry
        lax.fori_loop(0, (chunks - 2) // NBUF - 1, body3, 0)

        for j in range(chunks - NBUF, chunks):
            step(j, j % NBUF, first=False, last=(j == chunks - 1))
        s_desc(chunks - 2, (chunks - 2) % NBUF).wait()
        s_desc(chunks - 1, (chunks - 1) % NBUF).wait()

    return k


def kernel(x, pos, token_embed, pos_embed):
    b, l = x.shape
    n = b * l
    pos_vocab = pos_embed.shape[0]
    vocab = token_embed.shape[0]
    vocab_main = (vocab // 128) * 128
    n_tail = vocab - vocab_main
    pack_call, half = _tc_pack_table(vocab_main)
    packed = pack_call(token_embed.T, token_embed.T)
    tok_lin = packed.reshape(packed.shape[0] * PACK, DIM)
    small = jnp.concatenate(
        [pos_embed.reshape(pos_vocab * DIM),
         token_embed[vocab_main:].reshape(n_tail * DIM)])
    out = _sc_embed_add(n, pos_vocab, vocab_main, n_tail, half)(
        x.reshape(n), pos.reshape(n), tok_lin, small)
    out128 = lax.optimization_barrier(out.reshape(n * DIM // 128, 128))
    return out128.reshape(b, l, DIM)


# speedup vs baseline: 1.4933x; 1.4933x over previous
import functools

import jax
import jax.numpy as jnp
from jax import lax
from jax.experimental import pallas as pl
from jax.experimental.pallas import tpu as pltpu
from jax.experimental.pallas import tpu_sc as plsc

DIM = 64
LANES = 16
NUM_CORES = 2
NUM_SUBCORES = 16
NUM_WORKERS = NUM_CORES * NUM_SUBCORES

CHUNK = 128
NBUF = 3


def _sc_embed_add(n_rows, pos_vocab):
    assert n_rows % (NUM_WORKERS * CHUNK) == 0
    rows_per_w = n_rows // NUM_WORKERS
    chunks = rows_per_w // CHUNK
    assert chunks >= 5 and (chunks - 2) % NBUF == 0

    mesh = plsc.VectorSubcoreMesh(
        core_axis_name="c", subcore_axis_name="s",
        num_cores=NUM_CORES, num_subcores=NUM_SUBCORES)

    @functools.partial(
        pl.kernel,
        out_type=jax.ShapeDtypeStruct((n_rows, DIM), jnp.float32),
        mesh=mesh,
        compiler_params=pltpu.CompilerParams(use_tc_tiling_on_sc=False,
                                             needs_layout_passes=False),
        scratch_types=(
            [pltpu.VMEM((rows_per_w,), jnp.int32),
             pltpu.VMEM((rows_per_w,), jnp.int32),
             pltpu.VMEM((pos_vocab * DIM,), jnp.float32)]
            + [pltpu.VMEM((CHUNK, DIM), jnp.float32) for _ in range(NBUF)]
            + [pltpu.SemaphoreType.DMA for _ in range(2 * NBUF)]
        ),
    )
    def k(x_hbm, pos_hbm, tok_hbm, posemb_hbm, out_hbm,
          idx_t, idx_p, pos_tab, b0, b1, b2, g0, g1, g2, s0, s1, s2):
        wid = lax.axis_index("s") * NUM_CORES + lax.axis_index("c")
        base = wid * rows_per_w
        pltpu.sync_copy(x_hbm.at[pl.ds(base, rows_per_w)], idx_t)
        pltpu.sync_copy(pos_hbm.at[pl.ds(base, rows_per_w)], idx_p)
        pltpu.sync_copy(posemb_hbm, pos_tab)

        bufs = (b0, b1, b2)
        gsems = (g0, g1, g2)
        ssems = (s0, s1, s2)
        iota = lax.iota(jnp.int32, LANES)

        def g_desc(c, b):
            src = tok_hbm.at[idx_t.at[pl.ds(c * CHUNK, CHUNK)]]
            return pltpu.make_async_copy(src, bufs[b], gsems[b])

        def s_desc(c, b):
            dst = out_hbm.at[pl.ds(base + c * CHUNK, CHUNK)]
            return pltpu.make_async_copy(bufs[b], dst, ssems[b])

        zeros = jnp.zeros((LANES,), jnp.int32)

        def add_chunk(c, b):
            buf = bufs[b]

            @plsc.parallel_loop(0, CHUNK, step=1, unroll=2)
            def _row(r):
                pidx = plsc.load_gather(idx_p, [zeros + (c * CHUNK + r)])
                rbase = pidx * DIM
                for q in range(DIM // LANES):
                    addr = rbase + (iota + q * LANES)
                    pv = plsc.load_gather(pos_tab, [addr])
                    sl = pl.ds(q * LANES, LANES)
                    buf[r, sl] = buf[r, sl] + pv

        def step(j, b, first, last):
            nxt = (b + 1) % NBUF
            if not first:
                s_desc(j - 2, nxt).wait()
            if not last:
                g_desc(j + 1, nxt).start()
            g_desc(j, b).wait()
            add_chunk(j, b)
            s_desc(j, b).start()

        g_desc(0, 0).start()
        step(0, 0, first=True, last=False)
        step(1, 1, first=True, last=False)

        def body3(t, carry):
            for kk in range(NBUF):
                j = NBUF * t + (2 + kk)
                step(j, (2 + kk) % NBUF, first=False, last=False)
            return carry
        lax.fori_loop(0, (chunks - 2) // NBUF - 1, body3, 0)

        for j in range(chunks - NBUF, chunks):
            step(j, j % NBUF, first=False, last=(j == chunks - 1))
        s_desc(chunks - 2, (chunks - 2) % NBUF).wait()
        s_desc(chunks - 1, (chunks - 1) % NBUF).wait()

    return k


def kernel(x, pos, token_embed, pos_embed):
    b, l = x.shape
    n = b * l
    pos_vocab = pos_embed.shape[0]
    out = _sc_embed_add(n, pos_vocab)(
        x.reshape(n), pos.reshape(n), token_embed,
        pos_embed.reshape(pos_vocab * DIM))
    return out.reshape(b, l, DIM)


# speedup vs baseline: 1.5418x; 1.0325x over previous
import functools

import jax
import jax.numpy as jnp
from jax import lax
from jax.experimental import pallas as pl
from jax.experimental.pallas import tpu as pltpu
from jax.experimental.pallas import tpu_sc as plsc

DIM = 64
LANES = 16
NUM_CORES = 2
NUM_SUBCORES = 16
NUM_WORKERS = NUM_CORES * NUM_SUBCORES

CHUNK = 128
NBUF = 4


def _sc_embed_add(n_rows, pos_vocab):
    assert n_rows % (NUM_WORKERS * CHUNK) == 0
    rows_per_w = n_rows // NUM_WORKERS
    chunks = rows_per_w // CHUNK
    assert chunks >= 6 and (chunks - 4) % NBUF == 0

    mesh = plsc.VectorSubcoreMesh(
        core_axis_name="c", subcore_axis_name="s",
        num_cores=NUM_CORES, num_subcores=NUM_SUBCORES)

    @functools.partial(
        pl.kernel,
        out_type=jax.ShapeDtypeStruct((n_rows, DIM), jnp.float32),
        mesh=mesh,
        compiler_params=pltpu.CompilerParams(use_tc_tiling_on_sc=False,
                                             needs_layout_passes=False),
        scratch_types=(
            [pltpu.VMEM((rows_per_w,), jnp.int32),
             pltpu.VMEM((rows_per_w,), jnp.int32),
             pltpu.VMEM((pos_vocab * DIM,), jnp.float32)]
            + [pltpu.VMEM((CHUNK, DIM), jnp.float32) for _ in range(NBUF)]
            + [pltpu.SemaphoreType.DMA for _ in range(2 * NBUF)]
        ),
    )
    def k(x_hbm, pos_hbm, tok_hbm, posemb_hbm, out_hbm,
          idx_t, idx_p, pos_tab, b0, b1, b2, b3,
          g0, g1, g2, g3, s0, s1, s2, s3):
        wid = lax.axis_index("s") * NUM_CORES + lax.axis_index("c")
        base = wid * rows_per_w
        pltpu.sync_copy(x_hbm.at[pl.ds(base, rows_per_w)], idx_t)
        pltpu.sync_copy(pos_hbm.at[pl.ds(base, rows_per_w)], idx_p)
        pltpu.sync_copy(posemb_hbm, pos_tab)

        bufs = (b0, b1, b2, b3)
        gsems = (g0, g1, g2, g3)
        ssems = (s0, s1, s2, s3)
        iota = lax.iota(jnp.int32, LANES)

        def g_desc(c, b):
            src = tok_hbm.at[idx_t.at[pl.ds(c * CHUNK, CHUNK)]]
            return pltpu.make_async_copy(src, bufs[b], gsems[b])

        def s_desc(c, b):
            dst = out_hbm.at[pl.ds(base + c * CHUNK, CHUNK)]
            return pltpu.make_async_copy(bufs[b], dst, ssems[b])

        zeros = jnp.zeros((LANES,), jnp.int32)

        def add_chunk(c, b):
            buf = bufs[b]

            @plsc.parallel_loop(0, CHUNK, step=1, unroll=2)
            def _row(r):
                pidx = plsc.load_gather(idx_p, [zeros + (c * CHUNK + r)])
                rbase = pidx * DIM
                for q in range(DIM // LANES):
                    addr = rbase + (iota + q * LANES)
                    pv = plsc.load_gather(pos_tab, [addr])
                    sl = pl.ds(q * LANES, LANES)
                    buf[r, sl] = buf[r, sl] + pv

        def step(j, b, first, last):
            tgt = (b + 2) % NBUF
            if not first:
                s_desc(j - 2, tgt).wait()
            if not last:
                g_desc(j + 2, tgt).start()
            g_desc(j, b).wait()
            add_chunk(j, b)
            s_desc(j, b).start()

        g_desc(0, 0).start()
        g_desc(1, 1).start()
        step(0, 0, first=True, last=False)
        step(1, 1, first=True, last=False)

        def body4(t, carry):
            for kk in range(NBUF):
                j = NBUF * t + (2 + kk)
                step(j, (2 + kk) % NBUF, first=False, last=False)
            return carry
        lax.fori_loop(0, (chunks - 4) // NBUF, body4, 0)

        for j in range(chunks - 2, chunks):
            step(j, j % NBUF, first=False, last=True)
        s_desc(chunks - 2, (chunks - 2) % NBUF).wait()
        s_desc(chunks - 1, (chunks - 1) % NBUF).wait()

    return k


def kernel(x, pos, token_embed, pos_embed):
    b, l = x.shape
    n = b * l
    pos_vocab = pos_embed.shape[0]
    out = _sc_embed_add(n, pos_vocab)(
        x.reshape(n), pos.reshape(n), token_embed,
        pos_embed.reshape(pos_vocab * DIM))
    return out.reshape(b, l, DIM)
